# baseline (speedup 1.0000x reference)
import jax
import jax.numpy as jnp
from jax import lax
from jax.experimental import pallas as pl
from jax.experimental.pallas import tpu as pltpu

_F = 4
_H = 16
_BLOCK = 262144


def _mlp_t_body(x_ref, wt_ref, b_ref, o_ref):
    y = lax.dot_general(
        jnp.transpose(wt_ref[...]), x_ref[...],
        dimension_numbers=(((1,), (0,)), ((), ())),
        preferred_element_type=jnp.float32,
    )
    bcol = jnp.transpose(b_ref[...])
    o_ref[...] = jnp.maximum(y + bcol, 0.0)


def kernel(x, w1, b1):
    B = x.shape[0]
    bt = _BLOCK
    if B % bt:
        Bp = -(-B // bt) * bt
        x = jnp.pad(x, ((0, Bp - B), (0, 0)))
    else:
        Bp = B

    xt = x.T
    wt = w1.T

    out_t = pl.pallas_call(
        _mlp_t_body,
        out_shape=jax.ShapeDtypeStruct((_H, Bp), jnp.float32),
        grid=(Bp // bt,),
        in_specs=[
            pl.BlockSpec((_F, bt), lambda i: (0, i)),
            pl.BlockSpec((_F, _H), lambda i: (0, 0)),
            pl.BlockSpec((1, _H), lambda i: (0, 0)),
        ],
        out_specs=pl.BlockSpec((_H, bt), lambda i: (0, i)),
        compiler_params=pltpu.CompilerParams(
            dimension_semantics=("parallel",),
        ),
    )(xt, wt, b1)

    out = out_t.T
    return out if Bp == B else out[:B]
